# Initial kernel scaffold; baseline (speedup 1.0000x reference)
import jax
import jax.numpy as jnp
from jax.experimental import pallas as pl


def kernel(fp, sg, style_codes, mask_codes, bn_weight, bn_bias, conv_gamma_w, conv_gamma_b, conv_beta_w, conv_beta_b, fc_w, fc_b):
    raise NotImplementedError("write your pallas kernel here")


# speedup vs baseline: 11.0481x; 11.0481x over previous
import jax
import jax.numpy as jnp
from jax.experimental import pallas as pl

_B, _C, _H, _W = 4, 96, 224, 224
_NREG = 8
_SL = 256
_TH = 32
_EPS = 1e-5


def _pre_kernel(fp_ref, sg_ref, s_ref, idx_ref, ap_ref):
    b = pl.program_id(0)
    t = pl.program_id(1)

    @pl.when(jnp.logical_and(b == 0, t == 0))
    def _():
        s_ref[...] = jnp.zeros(s_ref.shape, s_ref.dtype)

    @pl.when(t == 0)
    def _():
        ap_ref[...] = jnp.zeros(ap_ref.shape, ap_ref.dtype)

    x = fp_ref[0]
    s_ref[0] += jnp.sum(x, axis=1)
    s_ref[1] += jnp.sum(x * x, axis=1)

    sgb = sg_ref[0]
    m = sgb > 0.5
    idx = jnp.full((_TH, _W), _NREG, jnp.int32)
    for j in range(_NREG):
        idx = jnp.where(m[j], j, idx)
    idx_ref[0] = idx
    ap_ref[0] += jnp.sum(m.astype(jnp.float32), axis=1)


def _mu_kernel(style_ref, mc_ref, ap_ref, fcT_ref, fcb_ref, w2_ref, g_ref):
    style = style_ref[...]
    mean_codes = jnp.mean(style, axis=1)
    area = jnp.sum(ap_ref[...], axis=-1)
    mus = []
    for j in range(_NREG):
        selj = jnp.where(mc_ref[:, j:j + 1] == 1, style[:, j, :], style[:, _NREG, :])
        inpj = jnp.where(area[:, j:j + 1] > 0, selj, mean_codes)
        muj = jnp.dot(inpj, fcT_ref[j], preferred_element_type=jnp.float32)
        muj = jnp.maximum(muj + fcb_ref[j:j + 1, :], 0.0)
        mus.append(muj)
    mu = jnp.concatenate(mus, axis=0)
    g_ref[...] = jnp.dot(mu, w2_ref[...], preferred_element_type=jnp.float32)


def _main_kernel(fp_ref, idxp_ref, big_ref, s_ref, bnw_ref, bnb_ref,
                 gb_ref, bb_ref, out_ref):
    t = pl.program_id(1)
    n = float(_B * _H * _W)
    mean = jnp.sum(s_ref[0], axis=-1, keepdims=True) / n
    ex2 = jnp.sum(s_ref[1], axis=-1, keepdims=True) / n
    var = ex2 - mean * mean
    scale = bnw_ref[...] * jax.lax.rsqrt(var + _EPS)
    shift = bnb_ref[...] - mean * scale

    big = big_ref[0]
    e = (jax.lax.broadcasted_iota(jnp.int32, (_C, 128), 1) == 108
         ).astype(jnp.float32)
    blocks = []
    for k in range(2):
        bg = big[2 * _C * k:2 * _C * k + _C] + e * (1.0 + gb_ref[...])
        bb = big[2 * _C * k + _C:2 * _C * (k + 1)] + e * bb_ref[...]
        blocks.append(bg * scale)
        blocks.append(bb + shift * bg)
    bigmod = jnp.concatenate(blocks, axis=0)

    A = idxp_ref[0, pl.ds(t * _TH, _TH + 2), :]
    J = jax.lax.broadcasted_iota(jnp.int32, (27, _W), 0) % 9
    sent = jnp.concatenate([jnp.ones((1, _W), jnp.float32),
                            jnp.zeros((128 - 109, _W), jnp.float32)], axis=0)

    def strip(p):
        row = jax.lax.slice(A, (p, 0), (p + 1, _W + 2))
        parts = [jnp.broadcast_to(jax.lax.slice(row, (0, dx), (1, dx + _W)),
                                  (9, _W)) for dx in range(3)]
        return (jnp.concatenate(parts, axis=0) == J).astype(jnp.float32)

    s0, s1 = strip(0), strip(1)
    for i in range(_TH // 2):
        s2, s3 = strip(2 * i + 2), strip(2 * i + 3)
        xq = jnp.concatenate([s0, s1, s2, s3, sent], axis=0)
        gb2 = jax.lax.dot_general(bigmod, xq, (((1,), (0,)), ((), ())),
                                  preferred_element_type=jnp.float32)
        for k in range(2):
            r = 2 * i + k
            gmul = gb2[2 * _C * k:2 * _C * k + _C]
            badd = gb2[2 * _C * k + _C:2 * _C * (k + 1)]
            out_ref[0, :, r, :] = fp_ref[0, :, r, :] * gmul + badd
        s0, s1 = s2, s3


def kernel(fp, sg, style_codes, mask_codes, bn_weight, bn_bias,
           conv_gamma_w, conv_gamma_b, conv_beta_w, conv_beta_b, fc_w, fc_b):
    f32 = jnp.float32
    nh = _H // _TH

    s, idx, ap = pl.pallas_call(
        _pre_kernel,
        grid=(_B, nh),
        in_specs=[
            pl.BlockSpec((1, _C, _TH, _W), lambda b, t: (b, 0, t, 0)),
            pl.BlockSpec((1, _NREG, _TH, _W), lambda b, t: (b, 0, t, 0)),
        ],
        out_specs=[
            pl.BlockSpec((2, _C, _W), lambda b, t: (0, 0, 0)),
            pl.BlockSpec((1, _TH, _W), lambda b, t: (b, t, 0)),
            pl.BlockSpec((1, _NREG, _W), lambda b, t: (b, 0, 0)),
        ],
        out_shape=[
            jax.ShapeDtypeStruct((2, _C, _W), f32),
            jax.ShapeDtypeStruct((_B, _H, _W), jnp.int32),
            jax.ShapeDtypeStruct((_B, _NREG, _W), f32),
        ],
    )(fp, sg)

    idxp = jnp.pad(idx, ((0, 0), (1, 1), (1, 1)), constant_values=_NREG)

    wg = conv_gamma_w.transpose(1, 2, 3, 0).reshape(_SL, 9, _C)
    wb = conv_beta_w.transpose(1, 2, 3, 0).reshape(_SL, 9, _C)
    w2 = jnp.concatenate([wg, wb], axis=-1).reshape(_SL, 9 * 2 * _C)
    fc_wT = jnp.swapaxes(fc_w, 1, 2)

    g = pl.pallas_call(
        _mu_kernel,
        out_shape=jax.ShapeDtypeStruct((_NREG * _B, 9 * 2 * _C), f32),
    )(style_codes, mask_codes.astype(jnp.int32), ap, fc_wT, fc_b, w2)

    g4 = g.reshape(_NREG, _B, 9, 2 * _C).transpose(1, 3, 2, 0)
    g4 = jnp.pad(g4, ((0, 0), (0, 0), (0, 0), (0, 1)))
    gt = g4.reshape(_B, 2 * _C, 81)
    big = jnp.concatenate([
        jnp.pad(gt, ((0, 0), (0, 0), (0, 128 - 81))),
        jnp.pad(gt, ((0, 0), (0, 0), (27, 128 - 108))),
    ], axis=1)

    out = pl.pallas_call(
        _main_kernel,
        grid=(_B, nh),
        in_specs=[
            pl.BlockSpec((1, _C, _TH, _W), lambda b, t: (b, 0, t, 0)),
            pl.BlockSpec((1, _H + 2, _W + 2), lambda b, t: (b, 0, 0)),
            pl.BlockSpec((1, 4 * _C, 128), lambda b, t: (b, 0, 0)),
            pl.BlockSpec((2, _C, _W), lambda b, t: (0, 0, 0)),
            pl.BlockSpec((_C, 1), lambda b, t: (0, 0)),
            pl.BlockSpec((_C, 1), lambda b, t: (0, 0)),
            pl.BlockSpec((_C, 1), lambda b, t: (0, 0)),
            pl.BlockSpec((_C, 1), lambda b, t: (0, 0)),
        ],
        out_specs=pl.BlockSpec((1, _C, _TH, _W), lambda b, t: (b, 0, t, 0)),
        out_shape=jax.ShapeDtypeStruct((_B, _C, _H, _W), f32),
    )(fp, idxp, big, s, bn_weight.reshape(_C, 1), bn_bias.reshape(_C, 1),
      conv_gamma_b.reshape(_C, 1), conv_beta_b.reshape(_C, 1))
    return out


# speedup vs baseline: 11.1876x; 1.0126x over previous
import jax
import jax.numpy as jnp
from jax.experimental import pallas as pl

_B, _C, _H, _W = 4, 96, 224, 224
_NREG = 8
_SL = 256
_TH = 56
_EPS = 1e-5
_HP, _WP = _H + 16, _W + 160


def _pre_kernel(fp_ref, sg_ref, s_ref, idx_ref, ap_ref):
    b = pl.program_id(0)
    t = pl.program_id(1)

    @pl.when(jnp.logical_and(b == 0, t == 0))
    def _():
        s_ref[...] = jnp.zeros(s_ref.shape, s_ref.dtype)

    @pl.when(t == 0)
    def _():
        ap_ref[...] = jnp.zeros(ap_ref.shape, ap_ref.dtype)

    x = fp_ref[0]
    s_ref[0] += jnp.sum(x, axis=1)
    s_ref[1] += jnp.sum(x * x, axis=1)

    @pl.when(t == 0)
    def _():
        idx_ref[...] = jnp.full(idx_ref.shape, _NREG, jnp.int32)

    sgb = sg_ref[0]
    m = sgb > 0.5
    idx = jnp.full((_TH, _W), _NREG, jnp.int32)
    for j in range(_NREG):
        idx = jnp.where(m[j], j, idx)
    idx_ref[0, pl.ds(8 + t * _TH, _TH), 128:128 + _W] = idx
    ap_ref[0] += jnp.sum(m.astype(jnp.float32), axis=1)


def _mu_kernel(style_ref, mc_ref, ap_ref, fcT_ref, fcb_ref, w2_ref, g_ref):
    style = style_ref[...]
    mean_codes = jnp.mean(style, axis=1)
    area = jnp.sum(ap_ref[...], axis=-1)
    mus = []
    for j in range(_NREG):
        selj = jnp.where(mc_ref[:, j:j + 1] == 1, style[:, j, :], style[:, _NREG, :])
        inpj = jnp.where(area[:, j:j + 1] > 0, selj, mean_codes)
        muj = jnp.dot(inpj, fcT_ref[j], preferred_element_type=jnp.float32)
        muj = jnp.maximum(muj + fcb_ref[j:j + 1, :], 0.0)
        mus.append(muj)
    mu = jnp.concatenate(mus, axis=0)
    g_ref[...] = jnp.dot(mu, w2_ref[...], preferred_element_type=jnp.float32)


def _main_kernel(fp_ref, idxp_ref, big_ref, s_ref, bnw_ref, bnb_ref,
                 gb_ref, bb_ref, out_ref):
    t = pl.program_id(1)
    n = float(_B * _H * _W)
    mean = jnp.sum(s_ref[0], axis=-1, keepdims=True) / n
    ex2 = jnp.sum(s_ref[1], axis=-1, keepdims=True) / n
    var = ex2 - mean * mean
    scale = bnw_ref[...] * jax.lax.rsqrt(var + _EPS)
    shift = bnb_ref[...] - mean * scale

    big = big_ref[0]
    e = (jax.lax.broadcasted_iota(jnp.int32, (_C, 128), 1) == 108
         ).astype(jnp.float32)
    blocks = []
    for k in range(2):
        bg = big[2 * _C * k:2 * _C * k + _C] + e * (1.0 + gb_ref[...])
        bb = big[2 * _C * k + _C:2 * _C * (k + 1)] + e * bb_ref[...]
        blocks.append(bg * scale)
        blocks.append(bb + shift * bg)
    bigmod = jnp.concatenate(blocks, axis=0)

    A = idxp_ref[0, pl.ds(t * _TH, _TH + 16), :]
    J = jax.lax.broadcasted_iota(jnp.int32, (27, _W), 0) % 9
    sent = jnp.concatenate([jnp.ones((1, _W), jnp.float32),
                            jnp.zeros((128 - 109, _W), jnp.float32)], axis=0)

    def strip(p):
        row = jax.lax.slice(A, (p + 7, 0), (p + 8, _WP))
        parts = [jnp.broadcast_to(
            jax.lax.slice(row, (0, 127 + dx), (1, 127 + dx + _W)), (9, _W))
            for dx in range(3)]
        return (jnp.concatenate(parts, axis=0) == J).astype(jnp.float32)

    s0, s1 = strip(0), strip(1)
    for i in range(_TH // 2):
        s2, s3 = strip(2 * i + 2), strip(2 * i + 3)
        xq = jnp.concatenate([s0, s1, s2, s3, sent], axis=0)
        gb2 = jax.lax.dot_general(bigmod, xq, (((1,), (0,)), ((), ())),
                                  preferred_element_type=jnp.float32)
        for k in range(2):
            r = 2 * i + k
            gmul = gb2[2 * _C * k:2 * _C * k + _C]
            badd = gb2[2 * _C * k + _C:2 * _C * (k + 1)]
            out_ref[0, :, r, :] = fp_ref[0, :, r, :] * gmul + badd
        s0, s1 = s2, s3


def kernel(fp, sg, style_codes, mask_codes, bn_weight, bn_bias,
           conv_gamma_w, conv_gamma_b, conv_beta_w, conv_beta_b, fc_w, fc_b):
    f32 = jnp.float32
    nh = _H // _TH

    s, idxp, ap = pl.pallas_call(
        _pre_kernel,
        grid=(_B, nh),
        in_specs=[
            pl.BlockSpec((1, _C, _TH, _W), lambda b, t: (b, 0, t, 0)),
            pl.BlockSpec((1, _NREG, _TH, _W), lambda b, t: (b, 0, t, 0)),
        ],
        out_specs=[
            pl.BlockSpec((2, _C, _W), lambda b, t: (0, 0, 0)),
            pl.BlockSpec((1, _HP, _WP), lambda b, t: (b, 0, 0)),
            pl.BlockSpec((1, _NREG, _W), lambda b, t: (b, 0, 0)),
        ],
        out_shape=[
            jax.ShapeDtypeStruct((2, _C, _W), f32),
            jax.ShapeDtypeStruct((_B, _HP, _WP), jnp.int32),
            jax.ShapeDtypeStruct((_B, _NREG, _W), f32),
        ],
    )(fp, sg)

    wg = conv_gamma_w.transpose(1, 2, 3, 0).reshape(_SL, 9, _C)
    wb = conv_beta_w.transpose(1, 2, 3, 0).reshape(_SL, 9, _C)
    w2 = jnp.concatenate([wg, wb], axis=-1).reshape(_SL, 9 * 2 * _C)
    fc_wT = jnp.swapaxes(fc_w, 1, 2)

    g = pl.pallas_call(
        _mu_kernel,
        out_shape=jax.ShapeDtypeStruct((_NREG * _B, 9 * 2 * _C), f32),
    )(style_codes, mask_codes.astype(jnp.int32), ap, fc_wT, fc_b, w2)

    g4 = g.reshape(_NREG, _B, 9, 2 * _C).transpose(1, 3, 2, 0)
    g4 = jnp.pad(g4, ((0, 0), (0, 0), (0, 0), (0, 1)))
    gt = g4.reshape(_B, 2 * _C, 81)
    big = jnp.concatenate([
        jnp.pad(gt, ((0, 0), (0, 0), (0, 128 - 81))),
        jnp.pad(gt, ((0, 0), (0, 0), (27, 128 - 108))),
    ], axis=1)

    out = pl.pallas_call(
        _main_kernel,
        grid=(_B, nh),
        in_specs=[
            pl.BlockSpec((1, _C, _TH, _W), lambda b, t: (b, 0, t, 0)),
            pl.BlockSpec((1, _HP, _WP), lambda b, t: (b, 0, 0)),
            pl.BlockSpec((1, 4 * _C, 128), lambda b, t: (b, 0, 0)),
            pl.BlockSpec((2, _C, _W), lambda b, t: (0, 0, 0)),
            pl.BlockSpec((_C, 1), lambda b, t: (0, 0)),
            pl.BlockSpec((_C, 1), lambda b, t: (0, 0)),
            pl.BlockSpec((_C, 1), lambda b, t: (0, 0)),
            pl.BlockSpec((_C, 1), lambda b, t: (0, 0)),
        ],
        out_specs=pl.BlockSpec((1, _C, _TH, _W), lambda b, t: (b, 0, t, 0)),
        out_shape=jax.ShapeDtypeStruct((_B, _C, _H, _W), f32),
    )(fp, idxp, big, s, bn_weight.reshape(_C, 1), bn_bias.reshape(_C, 1),
      conv_gamma_b.reshape(_C, 1), conv_beta_b.reshape(_C, 1))
    return out


# speedup vs baseline: 11.9244x; 1.0659x over previous
import jax
import jax.numpy as jnp
from jax.experimental import pallas as pl

_B, _C, _H, _W = 4, 96, 224, 224
_NREG = 8
_SL = 256
_TH = 56
_EPS = 1e-5
_HP, _WP = _H + 16, _W + 160


def _pre_kernel(fp_ref, sg_ref, style_ref, mc_ref, fcT_ref, fcb_ref, w2_ref,
                s_ref, idx_ref, ap_ref, g_ref):
    b = pl.program_id(0)
    t = pl.program_id(1)
    nh = pl.num_programs(1)

    @pl.when(jnp.logical_and(b == 0, t == 0))
    def _():
        s_ref[...] = jnp.zeros(s_ref.shape, s_ref.dtype)
        ap_ref[...] = jnp.zeros(ap_ref.shape, ap_ref.dtype)

    x = fp_ref[0]
    s_ref[0] += jnp.sum(x, axis=1)
    s_ref[1] += jnp.sum(x * x, axis=1)

    @pl.when(t == 0)
    def _():
        idx_ref[...] = jnp.full(idx_ref.shape, _NREG, jnp.int32)

    sgb = sg_ref[0]
    m = sgb > 0.5
    idx = jnp.full((_TH, _W), _NREG, jnp.int32)
    for j in range(_NREG):
        idx = jnp.where(m[j], j, idx)
    idx_ref[0, pl.ds(8 + t * _TH, _TH), 128:128 + _W] = idx
    part = jnp.sum(m.astype(jnp.float32), axis=1)
    bsel = jax.lax.broadcasted_iota(jnp.int32, (_B, _NREG, _W), 0) == b
    ap_ref[...] += jnp.where(bsel, part[None], 0.0)

    @pl.when(jnp.logical_and(b == _B - 1, t == nh - 1))
    def _():
        style = style_ref[...]
        mean_codes = jnp.mean(style, axis=1)
        area = jnp.sum(ap_ref[...], axis=-1)
        mus = []
        for j in range(_NREG):
            selj = jnp.where(mc_ref[:, j:j + 1] == 1,
                             style[:, j, :], style[:, _NREG, :])
            inpj = jnp.where(area[:, j:j + 1] > 0, selj, mean_codes)
            muj = jnp.dot(inpj, fcT_ref[j], preferred_element_type=jnp.float32)
            muj = jnp.maximum(muj + fcb_ref[j:j + 1, :], 0.0)
            mus.append(muj)
        mu = jnp.concatenate(mus, axis=0)
        g_ref[...] = jnp.dot(mu, w2_ref[...], preferred_element_type=jnp.float32)


def _main_kernel(fp_ref, idxp_ref, big_ref, s_ref, bnw_ref, bnb_ref,
                 gb_ref, bb_ref, out_ref):
    t = pl.program_id(1)
    n = float(_B * _H * _W)
    mean = jnp.sum(s_ref[0], axis=-1, keepdims=True) / n
    ex2 = jnp.sum(s_ref[1], axis=-1, keepdims=True) / n
    var = ex2 - mean * mean
    scale = bnw_ref[...] * jax.lax.rsqrt(var + _EPS)
    shift = bnb_ref[...] - mean * scale

    big = big_ref[0]
    lane = jax.lax.broadcasted_iota(jnp.int32, (_C, 128), 1)
    blocks = []
    for k in range(2):
        e = jnp.logical_and(lane // 32 == k, lane % 32 < 9).astype(jnp.float32)
        bg = big[2 * _C * k:2 * _C * k + _C] + e * (1.0 + gb_ref[...])
        bb = big[2 * _C * k + _C:2 * _C * (k + 1)] + e * bb_ref[...]
        blocks.append(bg * scale)
        blocks.append(bb + shift * bg)
    bigmod_a = jnp.concatenate(blocks, axis=0).astype(jnp.bfloat16)
    bigmod_b = jnp.concatenate([bigmod_a[:, 64:], bigmod_a[:, :64]], axis=1)

    A = idxp_ref[0, pl.ds(t * _TH, _TH + 16), :]
    srow = jax.lax.broadcasted_iota(jnp.int32, (32, _W), 0)
    dsel = srow // 9
    jsel = srow - 9 * dsel

    def strip(p):
        row = jax.lax.slice(A, (p + 7, 0), (p + 8, _WP))
        y = [jnp.broadcast_to(
            jax.lax.slice(row, (0, 127 + dx), (1, 127 + dx + _W)), (32, _W))
            for dx in range(3)]
        yy = jnp.where(dsel == 0, y[0], jnp.where(dsel == 1, y[1], y[2]))
        return (yy == jsel).astype(jnp.bfloat16)

    slots = [strip(0), strip(1), strip(2), strip(3)]
    npair = _TH // 2
    for i in range(npair):
        xq = jnp.concatenate(slots, axis=0)
        bm = bigmod_a if i % 2 == 0 else bigmod_b
        gb2 = jax.lax.dot_general(bm, xq, (((1,), (0,)), ((), ())),
                                  preferred_element_type=jnp.float32)
        for k in range(2):
            r = 2 * i + k
            gmul = gb2[2 * _C * k:2 * _C * k + _C]
            badd = gb2[2 * _C * k + _C:2 * _C * (k + 1)]
            out_ref[0, :, r, :] = fp_ref[0, :, r, :] * gmul + badd
        if i + 1 < npair:
            slots[(2 * i) % 4] = strip(2 * i + 4)
            slots[(2 * i + 1) % 4] = strip(2 * i + 5)


def kernel(fp, sg, style_codes, mask_codes, bn_weight, bn_bias,
           conv_gamma_w, conv_gamma_b, conv_beta_w, conv_beta_b, fc_w, fc_b):
    f32 = jnp.float32
    nh = _H // _TH

    wg = conv_gamma_w.transpose(1, 2, 3, 0).reshape(_SL, 9, _C)
    wb = conv_beta_w.transpose(1, 2, 3, 0).reshape(_SL, 9, _C)
    w2 = jnp.concatenate([wg, wb], axis=-1).reshape(_SL, 9 * 2 * _C)
    fc_wT = jnp.swapaxes(fc_w, 1, 2)

    s, idxp, ap, g = pl.pallas_call(
        _pre_kernel,
        grid=(_B, nh),
        in_specs=[
            pl.BlockSpec((1, _C, _TH, _W), lambda b, t: (b, 0, t, 0)),
            pl.BlockSpec((1, _NREG, _TH, _W), lambda b, t: (b, 0, t, 0)),
            pl.BlockSpec((_B, _NREG + 1, _SL), lambda b, t: (0, 0, 0)),
            pl.BlockSpec((_B, _NREG), lambda b, t: (0, 0)),
            pl.BlockSpec((_NREG, _SL, _SL), lambda b, t: (0, 0, 0)),
            pl.BlockSpec((_NREG, _SL), lambda b, t: (0, 0)),
            pl.BlockSpec((_SL, 9 * 2 * _C), lambda b, t: (0, 0)),
        ],
        out_specs=[
            pl.BlockSpec((2, _C, _W), lambda b, t: (0, 0, 0)),
            pl.BlockSpec((1, _HP, _WP), lambda b, t: (b, 0, 0)),
            pl.BlockSpec((_B, _NREG, _W), lambda b, t: (0, 0, 0)),
            pl.BlockSpec((_NREG * _B, 9 * 2 * _C), lambda b, t: (0, 0)),
        ],
        out_shape=[
            jax.ShapeDtypeStruct((2, _C, _W), f32),
            jax.ShapeDtypeStruct((_B, _HP, _WP), jnp.int32),
            jax.ShapeDtypeStruct((_B, _NREG, _W), f32),
            jax.ShapeDtypeStruct((_NREG * _B, 9 * 2 * _C), f32),
        ],
    )(fp, sg, style_codes, mask_codes.astype(jnp.int32), fc_wT, fc_b, w2)

    g4 = g.reshape(_NREG, _B, 9, 2 * _C).transpose(1, 3, 2, 0)
    g4 = jnp.pad(g4, ((0, 0), (0, 0), (0, 0), (0, 1)))
    gt = jnp.pad(g4.reshape(_B, 2 * _C, 3, 27),
                 ((0, 0), (0, 0), (0, 0), (0, 5))).reshape(_B, 2 * _C, 96)
    big = jnp.concatenate([
        jnp.pad(gt, ((0, 0), (0, 0), (0, 32))),
        jnp.pad(gt, ((0, 0), (0, 0), (32, 0))),
    ], axis=1)

    out = pl.pallas_call(
        _main_kernel,
        grid=(_B, nh),
        in_specs=[
            pl.BlockSpec((1, _C, _TH, _W), lambda b, t: (b, 0, t, 0)),
            pl.BlockSpec((1, _HP, _WP), lambda b, t: (b, 0, 0)),
            pl.BlockSpec((1, 4 * _C, 128), lambda b, t: (b, 0, 0)),
            pl.BlockSpec((2, _C, _W), lambda b, t: (0, 0, 0)),
            pl.BlockSpec((_C, 1), lambda b, t: (0, 0)),
            pl.BlockSpec((_C, 1), lambda b, t: (0, 0)),
            pl.BlockSpec((_C, 1), lambda b, t: (0, 0)),
            pl.BlockSpec((_C, 1), lambda b, t: (0, 0)),
        ],
        out_specs=pl.BlockSpec((1, _C, _TH, _W), lambda b, t: (b, 0, t, 0)),
        out_shape=jax.ShapeDtypeStruct((_B, _C, _H, _W), f32),
    )(fp, idxp, big, s, bn_weight.reshape(_C, 1), bn_bias.reshape(_C, 1),
      conv_gamma_b.reshape(_C, 1), conv_beta_b.reshape(_C, 1))
    return out
